# baseline (speedup 1.0000x reference)
import jax
import jax.numpy as jnp
from jax.experimental import pallas as pl
from jax.experimental.pallas import tpu as pltpu

B = 32
ZS = 128
CC = 16
N_OUT = 65536
N_CHUNKS = 8
CHUNK = N_OUT // N_CHUNKS


def _leaky(x):
    return jnp.where(x > 0, x, 0.2 * x)


def _fgl_kernel(studies_ref, tasks_ref, contrasts_ref, z_ref,
                Es_ref, Et_ref, Ec_ref,
                fc0_W_ref, fc0_b_ref, fc1_W_ref, fc1_b_ref, fc2_W_ref,
                fc2_b_ref, fc3_W_ref, fc3_b_ref, fc4_W_ref, fc4_b_ref,
                up0_W_ref, up0_b_ref, up1_W_ref, up1_b_ref, up2_W_ref,
                up2_b_ref, up3_W_ref, up3_b_ref, up4_W_ref, up4_b_ref,
                out_ref, y_ref):
    f32 = jnp.float32

    @pl.when(pl.program_id(0) == 0)
    def _compute_chain():
        def onehot(idx_ref, n):
            iota = jax.lax.broadcasted_iota(jnp.int32, (B, n), 1)
            return (iota == idx_ref[:, :]).astype(f32)

        se = onehot(studies_ref, Es_ref.shape[0]) @ Es_ref[:, :]
        te = onehot(tasks_ref, Et_ref.shape[0]) @ Et_ref[:, :]
        ce = onehot(contrasts_ref, Ec_ref.shape[0]) @ Ec_ref[:, :]
        cat3 = jnp.concatenate([se, te, ce], axis=1)

        c0 = se @ fc0_W_ref[:, :] + fc0_b_ref[:, :]
        c1 = jnp.concatenate([se, te], axis=1) @ fc1_W_ref[:, :] + fc1_b_ref[:, :]
        c2 = cat3 @ fc2_W_ref[:, :] + fc2_b_ref[:, :]
        c3 = cat3 @ fc3_W_ref[:, :] + fc3_b_ref[:, :]
        c4 = cat3 @ fc4_W_ref[:, :] + fc4_b_ref[:, :]

        x = z_ref[:, :]
        x = _leaky(jnp.concatenate([x, c0], axis=1) @ up0_W_ref[:, :] + up0_b_ref[:, :])
        x = _leaky(jnp.concatenate([x, c1], axis=1) @ up1_W_ref[:, :] + up1_b_ref[:, :])
        x = _leaky(jnp.concatenate([x, c2], axis=1) @ up2_W_ref[:, :] + up2_b_ref[:, :])
        x = _leaky(jnp.concatenate([x, c3], axis=1) @ up3_W_ref[:, :] + up3_b_ref[:, :])
        y = jnp.concatenate([x, c4], axis=1) @ up4_W_ref[:, :] + up4_b_ref[:, :]
        y_ref[:, :] = y

    out_ref[:, :] = jnp.broadcast_to(y_ref[:, :], (B, CHUNK))


def kernel(z, studies, tasks, contrasts, Es, Et, Ec,
           fc0_W, fc0_b, fc1_W, fc1_b, fc2_W, fc2_b, fc3_W, fc3_b,
           fc4_W, fc4_b, up0_W, up0_b, up1_W, up1_b, up2_W, up2_b,
           up3_W, up3_b, up4_W, up4_b, idx0, idx1, idx2, idx3, idx4):
    args = (
        studies.reshape(B, 1), tasks.reshape(B, 1), contrasts.reshape(B, 1),
        z, Es, Et, Ec,
        fc0_W, fc0_b.reshape(1, -1), fc1_W, fc1_b.reshape(1, -1),
        fc2_W, fc2_b.reshape(1, -1), fc3_W, fc3_b.reshape(1, -1),
        fc4_W, fc4_b.reshape(1, -1),
        up0_W, up0_b.reshape(1, -1), up1_W, up1_b.reshape(1, -1),
        up2_W, up2_b.reshape(1, -1), up3_W, up3_b.reshape(1, -1),
        up4_W, up4_b.reshape(1, -1),
    )
    out = pl.pallas_call(
        _fgl_kernel,
        grid=(N_CHUNKS,),
        in_specs=[pl.BlockSpec(a.shape, lambda i, n=a.ndim: (0,) * n)
                  for a in args],
        out_specs=pl.BlockSpec((B, CHUNK), lambda i: (0, i)),
        out_shape=jax.ShapeDtypeStruct((B, N_OUT), jnp.float32),
        scratch_shapes=[pltpu.VMEM((B, 1), jnp.float32)],
    )(*args)
    return out.reshape(B, N_OUT, 1)


# speedup vs baseline: 1.8451x; 1.8451x over previous
import jax
import jax.numpy as jnp
from jax.experimental import pallas as pl
from jax.experimental.pallas import tpu as pltpu

B = 32
ZS = 128
CC = 16
N_OUT = 65536
N_CHUNKS = 8
CHUNK = N_OUT // N_CHUNKS


def _leaky(x):
    return jnp.where(x > 0, x, 0.2 * x)


def _fgl_kernel(studies_ref, tasks_ref, contrasts_ref, z_ref,
                Es_ref, Et_ref, Ec_ref,
                fc0_W_ref, fc0_b_ref, fc1_W_ref, fc1_b_ref, fc2_W_ref,
                fc2_b_ref, fc3_W_ref, fc3_b_ref, fc4_W_ref, fc4_b_ref,
                up0_W_ref, up0_b_ref, up1_W_ref, up1_b_ref, up2_W_ref,
                up2_b_ref, up3_W_ref, up3_b_ref, up4_W_ref, up4_b_ref,
                out_ref, y_ref):
    f32 = jnp.float32

    @pl.when(pl.program_id(0) == 0)
    def _compute_chain():
        def onehot(idx_ref, n):
            iota = jax.lax.broadcasted_iota(jnp.int32, (B, n), 1)
            return (iota == idx_ref[:, :]).astype(f32)

        se = onehot(studies_ref, Es_ref.shape[0]) @ Es_ref[:, :]
        te = onehot(tasks_ref, Et_ref.shape[0]) @ Et_ref[:, :]
        ce = onehot(contrasts_ref, Ec_ref.shape[0]) @ Ec_ref[:, :]
        cat3 = jnp.concatenate([se, te, ce], axis=1)

        c0 = se @ fc0_W_ref[:, :] + fc0_b_ref[:, :]
        c1 = jnp.concatenate([se, te], axis=1) @ fc1_W_ref[:, :] + fc1_b_ref[:, :]
        c2 = cat3 @ fc2_W_ref[:, :] + fc2_b_ref[:, :]
        c3 = cat3 @ fc3_W_ref[:, :] + fc3_b_ref[:, :]
        c4 = cat3 @ fc4_W_ref[:, :] + fc4_b_ref[:, :]

        x = z_ref[:, :]
        x = _leaky(jnp.concatenate([x, c0], axis=1) @ up0_W_ref[:, :] + up0_b_ref[:, :])
        x = _leaky(jnp.concatenate([x, c1], axis=1) @ up1_W_ref[:, :] + up1_b_ref[:, :])
        x = _leaky(jnp.concatenate([x, c2], axis=1) @ up2_W_ref[:, :] + up2_b_ref[:, :])
        x = _leaky(jnp.concatenate([x, c3], axis=1) @ up3_W_ref[:, :] + up3_b_ref[:, :])
        y = jnp.concatenate([x, c4], axis=1) @ up4_W_ref[:, :] + up4_b_ref[:, :]
        y_ref[:, :] = y

    yv = y_ref[:, :]
    out_ref[:, :, :] = jnp.broadcast_to(yv[:, :, None], (B, CHUNK // 128, 128))


def kernel(z, studies, tasks, contrasts, Es, Et, Ec,
           fc0_W, fc0_b, fc1_W, fc1_b, fc2_W, fc2_b, fc3_W, fc3_b,
           fc4_W, fc4_b, up0_W, up0_b, up1_W, up1_b, up2_W, up2_b,
           up3_W, up3_b, up4_W, up4_b, idx0, idx1, idx2, idx3, idx4):
    args = (
        studies.reshape(B, 1), tasks.reshape(B, 1), contrasts.reshape(B, 1),
        z, Es, Et, Ec,
        fc0_W, fc0_b.reshape(1, -1), fc1_W, fc1_b.reshape(1, -1),
        fc2_W, fc2_b.reshape(1, -1), fc3_W, fc3_b.reshape(1, -1),
        fc4_W, fc4_b.reshape(1, -1),
        up0_W, up0_b.reshape(1, -1), up1_W, up1_b.reshape(1, -1),
        up2_W, up2_b.reshape(1, -1), up3_W, up3_b.reshape(1, -1),
        up4_W, up4_b.reshape(1, -1),
    )
    out = pl.pallas_call(
        _fgl_kernel,
        grid=(N_CHUNKS,),
        in_specs=[pl.BlockSpec(a.shape, lambda i, n=a.ndim: (0,) * n)
                  for a in args],
        out_specs=pl.BlockSpec((B, CHUNK // 128, 128), lambda i: (0, i, 0)),
        out_shape=jax.ShapeDtypeStruct((B, N_OUT // 128, 128), jnp.float32),
        scratch_shapes=[pltpu.VMEM((B, 1), jnp.float32)],
    )(*args)
    return out.reshape(B, N_OUT, 1)
